# baseline (speedup 1.0000x reference)
import functools

import jax
import jax.numpy as jnp
from jax import lax
from jax.experimental import pallas as pl
from jax.experimental.pallas import tpu as pltpu
from jax.experimental.pallas import tpu_sc as plsc

BATCH = 4096
SEQ = 200
D = 64
NUM_CLASSES = 5
VOCAB = 1000000

CPAD = 16
NB = 1024

NUM_CORES = 2
NUM_SUBCORES = 16
NUM_WORKERS = NUM_CORES * NUM_SUBCORES
B_PER_W = BATCH // NUM_WORKERS
IDX_PER_W = B_PER_W * SEQ
CHUNK = 128
N_CHUNKS = IDX_PER_W // CHUNK


def _tc_project(tT, Wp):
  def k(t_ref, w_ref, o_ref):
    o_ref[...] = jnp.dot(
        w_ref[...],
        t_ref[...],
        preferred_element_type=jnp.float32,
        precision=lax.Precision.HIGHEST,
    )

  return pl.pallas_call(
      k,
      grid=(pl.cdiv(VOCAB, NB),),
      in_specs=[
          pl.BlockSpec((D, NB), lambda i: (0, i)),
          pl.BlockSpec((CPAD, D), lambda i: (0, 0)),
      ],
      out_specs=pl.BlockSpec((CPAD, NB), lambda i: (0, i)),
      out_shape=jax.ShapeDtypeStruct((CPAD, VOCAB), jnp.float32),
  )(tT, Wp)


def _sc_pooled_sums(x1, proj):
  mesh = plsc.VectorSubcoreMesh(core_axis_name="c", subcore_axis_name="s")

  @functools.partial(
      pl.kernel,
      mesh=mesh,
      compiler_params=pltpu.CompilerParams(use_tc_tiling_on_sc=False),
      out_type=jax.ShapeDtypeStruct((BATCH, CPAD), jnp.float32),
      scratch_types=[
          pltpu.VMEM((IDX_PER_W,), jnp.int32),
          pltpu.VMEM((2, CHUNK, CPAD), jnp.float32),
          pltpu.VMEM((B_PER_W, CPAD), jnp.float32),
          pltpu.SemaphoreType.DMA,
          pltpu.SemaphoreType.DMA,
      ],
  )
  def sc_kernel(x_hbm, proj_hbm, out_hbm, idx_v, rows_v, acc_v, sem0, sem1):
    wid = lax.axis_index("s") * NUM_CORES + lax.axis_index("c")
    sems = (sem0, sem1)
    pltpu.sync_copy(x_hbm.at[pl.ds(wid * IDX_PER_W, IDX_PER_W)], idx_v)

    def zero_body(b, _):
      acc_v[b, pl.ds(0, CPAD)] = jnp.zeros((CPAD,), jnp.float32)
      return 0

    lax.fori_loop(0, B_PER_W, zero_body, 0)

    def issue(c, buf):
      pltpu.async_copy(
          proj_hbm.at[idx_v.at[pl.ds(c * CHUNK, CHUNK)]],
          rows_v.at[buf],
          sems[buf],
      )

    def wait(c, buf):
      pltpu.make_async_copy(
          proj_hbm.at[idx_v.at[pl.ds(c * CHUNK, CHUNK)]],
          rows_v.at[buf],
          sems[buf],
      ).wait()

    def reduce_chunk(c, buf):
      start = c * CHUNK
      b0 = start // SEQ
      s = jnp.minimum((b0 + 1) * SEQ - start, CHUNK)

      def seg_sum(lo, hi, row):
        def red_body(r, carry):
          return carry + rows_v[buf, r, pl.ds(0, CPAD)]

        acc = lax.fori_loop(lo, hi, red_body, jnp.zeros((CPAD,), jnp.float32))
        sl = pl.ds(0, CPAD)
        acc_v[row, sl] = acc_v[row, sl] + acc

      seg_sum(0, s, b0)
      seg_sum(s, CHUNK, b0 + 1)

    issue(0, 0)

    def pair_body(p, _):
      c0 = 2 * p
      issue(c0 + 1, 1)
      wait(c0, 0)
      reduce_chunk(c0, 0)

      @pl.when(p < N_CHUNKS // 2 - 1)
      def _():
        issue(c0 + 2, 0)

      wait(c0 + 1, 1)
      reduce_chunk(c0 + 1, 1)
      return 0

    lax.fori_loop(0, N_CHUNKS // 2, pair_body, 0)
    pltpu.sync_copy(acc_v, out_hbm.at[pl.ds(wid * B_PER_W, B_PER_W)])

  return sc_kernel(x1, proj)


def _tc_epilogue(sums, x, proj0, sel, b):
  def tc_kernel(sums_ref, x_ref, p0_ref, sel_ref, b_ref, out_ref):
    n0 = jnp.sum((x_ref[...] == 0).astype(jnp.float32), axis=1, keepdims=True)
    mean = (sums_ref[...] - n0 * p0_ref[...]) * (1.0 / SEQ)
    out_ref[...] = (
        jnp.dot(
            mean,
            sel_ref[...],
            preferred_element_type=jnp.float32,
            precision=lax.Precision.HIGHEST,
        )
        + b_ref[...]
    )

  return pl.pallas_call(
      tc_kernel,
      out_shape=jax.ShapeDtypeStruct((BATCH, NUM_CLASSES), jnp.float32),
  )(sums, x, proj0, sel, b)


def kernel(x, table, W, b):
  tT = jnp.swapaxes(table, 0, 1)
  Wp = jnp.zeros((CPAD, D), jnp.float32).at[:NUM_CLASSES].set(W)
  projT = _tc_project(tT, Wp)
  proj = jnp.swapaxes(projT, 0, 1)
  x1 = x.reshape(BATCH * SEQ)
  sums = _sc_pooled_sums(x1, proj)
  proj0 = lax.slice(proj, (0, 0), (1, CPAD))
  sel = jnp.eye(CPAD, NUM_CLASSES, dtype=jnp.float32)
  return _tc_epilogue(sums, x, proj0, sel, b.reshape(1, NUM_CLASSES))


# speedup vs baseline: 1.0279x; 1.0279x over previous
import functools

import jax
import jax.numpy as jnp
from jax import lax
from jax.experimental import pallas as pl
from jax.experimental.pallas import tpu as pltpu
from jax.experimental.pallas import tpu_sc as plsc

BATCH = 4096
SEQ = 200
D = 64
NUM_CLASSES = 5
VOCAB = 1000000

CPAD = 16
NB = 1024

NUM_CORES = 2
NUM_SUBCORES = 16
NUM_WORKERS = NUM_CORES * NUM_SUBCORES
B_PER_W = BATCH // NUM_WORKERS
IDX_PER_W = B_PER_W * SEQ
CHUNK = 128
N_CHUNKS = IDX_PER_W // CHUNK


def _tc_project(tT, Wp):
  def k(t_ref, w_ref, o_ref):
    o_ref[...] = jnp.dot(
        w_ref[...],
        t_ref[...],
        preferred_element_type=jnp.float32,
    )

  return pl.pallas_call(
      k,
      grid=(pl.cdiv(VOCAB, NB),),
      in_specs=[
          pl.BlockSpec((D, NB), lambda i: (0, i)),
          pl.BlockSpec((CPAD, D), lambda i: (0, 0)),
      ],
      out_specs=pl.BlockSpec((CPAD, NB), lambda i: (0, i)),
      out_shape=jax.ShapeDtypeStruct((CPAD, VOCAB), jnp.float32),
  )(tT, Wp)


def _sc_pooled_sums(x1, proj):
  mesh = plsc.VectorSubcoreMesh(core_axis_name="c", subcore_axis_name="s")

  @functools.partial(
      pl.kernel,
      mesh=mesh,
      compiler_params=pltpu.CompilerParams(use_tc_tiling_on_sc=False),
      out_type=jax.ShapeDtypeStruct((BATCH, CPAD), jnp.float32),
      scratch_types=[
          pltpu.VMEM((IDX_PER_W,), jnp.int32),
          pltpu.VMEM((2, CHUNK, CPAD), jnp.float32),
          pltpu.VMEM((B_PER_W, CPAD), jnp.float32),
          pltpu.SemaphoreType.DMA,
          pltpu.SemaphoreType.DMA,
      ],
  )
  def sc_kernel(x_hbm, proj_hbm, out_hbm, idx_v, rows_v, acc_v, sem0, sem1):
    wid = lax.axis_index("s") * NUM_CORES + lax.axis_index("c")
    sems = (sem0, sem1)
    pltpu.sync_copy(x_hbm.at[pl.ds(wid * IDX_PER_W, IDX_PER_W)], idx_v)

    def zero_body(b, _):
      acc_v[b, pl.ds(0, CPAD)] = jnp.zeros((CPAD,), jnp.float32)
      return 0

    lax.fori_loop(0, B_PER_W, zero_body, 0)

    def issue(c, buf):
      pltpu.async_copy(
          proj_hbm.at[idx_v.at[pl.ds(c * CHUNK, CHUNK)]],
          rows_v.at[buf],
          sems[buf],
      )

    def wait(c, buf):
      pltpu.make_async_copy(
          proj_hbm.at[idx_v.at[pl.ds(c * CHUNK, CHUNK)]],
          rows_v.at[buf],
          sems[buf],
      ).wait()

    def reduce_chunk(c, buf):
      start = c * CHUNK
      b0 = start // SEQ
      s = jnp.minimum((b0 + 1) * SEQ - start, CHUNK)

      def seg_sum(lo, hi, row):
        def red_body(r, carry):
          return carry + rows_v[buf, r, pl.ds(0, CPAD)]

        acc = lax.fori_loop(lo, hi, red_body, jnp.zeros((CPAD,), jnp.float32))
        sl = pl.ds(0, CPAD)
        acc_v[row, sl] = acc_v[row, sl] + acc

      seg_sum(0, s, b0)
      seg_sum(s, CHUNK, b0 + 1)

    issue(0, 0)

    def pair_body(p, _):
      c0 = 2 * p
      issue(c0 + 1, 1)
      wait(c0, 0)
      reduce_chunk(c0, 0)

      @pl.when(p < N_CHUNKS // 2 - 1)
      def _():
        issue(c0 + 2, 0)

      wait(c0 + 1, 1)
      reduce_chunk(c0 + 1, 1)
      return 0

    lax.fori_loop(0, N_CHUNKS // 2, pair_body, 0)
    pltpu.sync_copy(acc_v, out_hbm.at[pl.ds(wid * B_PER_W, B_PER_W)])

  return sc_kernel(x1, proj)


def _tc_epilogue(sums, x, proj0, sel, b):
  def tc_kernel(sums_ref, x_ref, p0_ref, sel_ref, b_ref, out_ref):
    n0 = jnp.sum((x_ref[...] == 0).astype(jnp.float32), axis=1, keepdims=True)
    mean = (sums_ref[...] - n0 * p0_ref[...]) * (1.0 / SEQ)
    out_ref[...] = (
        jnp.dot(
            mean,
            sel_ref[...],
            preferred_element_type=jnp.float32,
            precision=lax.Precision.HIGHEST,
        )
        + b_ref[...]
    )

  return pl.pallas_call(
      tc_kernel,
      out_shape=jax.ShapeDtypeStruct((BATCH, NUM_CLASSES), jnp.float32),
  )(sums, x, proj0, sel, b)


def kernel(x, table, W, b):
  tT = jnp.swapaxes(table, 0, 1)
  Wp = jnp.zeros((CPAD, D), jnp.float32).at[:NUM_CLASSES].set(W)
  projT = _tc_project(tT, Wp)
  proj = jnp.swapaxes(projT, 0, 1)
  x1 = x.reshape(BATCH * SEQ)
  sums = _sc_pooled_sums(x1, proj)
  proj0 = lax.slice(proj, (0, 0), (1, CPAD))
  sel = jnp.eye(CPAD, NUM_CLASSES, dtype=jnp.float32)
  return _tc_epilogue(sums, x, proj0, sel, b.reshape(1, NUM_CLASSES))


# speedup vs baseline: 1.8377x; 1.7878x over previous
import functools

import jax
import jax.numpy as jnp
from jax import lax
from jax.experimental import pallas as pl
from jax.experimental.pallas import tpu as pltpu
from jax.experimental.pallas import tpu_sc as plsc

BATCH = 4096
SEQ = 200
D = 64
NUM_CLASSES = 5

NUM_CORES = 2
NUM_SUBCORES = 16
NUM_WORKERS = NUM_CORES * NUM_SUBCORES
B_PER_W = BATCH // NUM_WORKERS
SEQ_PAD = 208
CHUNKS = ((0, 128), (128, 72))
LANES = 16
DV = D // LANES


def _sc_pooled_sums(x, table):
  mesh = plsc.VectorSubcoreMesh(core_axis_name="c", subcore_axis_name="s")

  @functools.partial(
      pl.kernel,
      mesh=mesh,
      compiler_params=pltpu.CompilerParams(use_tc_tiling_on_sc=False),
      out_type=jax.ShapeDtypeStruct((BATCH, D), jnp.float32),
      scratch_types=[
          pltpu.VMEM((B_PER_W, SEQ_PAD), jnp.int32),
          pltpu.VMEM((2, SEQ, D), jnp.float32),
          pltpu.VMEM((B_PER_W, D), jnp.float32),
          pltpu.SemaphoreType.DMA,
          pltpu.SemaphoreType.DMA,
      ],
  )
  def sc_kernel(x_hbm, table_hbm, out_hbm, idx_v, rows_v, acc_v, sem0, sem1):
    wid = lax.axis_index("s") * NUM_CORES + lax.axis_index("c")
    base = wid * B_PER_W
    sems = (sem0, sem1)
    pltpu.sync_copy(
        x_hbm.at[pl.ds(base, B_PER_W)],
        idx_v.at[:, pl.ds(0, SEQ)],
    )

    def issue(b, buf):
      for off, ln in CHUNKS:
        pltpu.async_copy(
            table_hbm.at[idx_v.at[b, pl.ds(off, ln)]],
            rows_v.at[buf, pl.ds(off, ln)],
            sems[buf],
        )

    def wait(b, buf):
      for off, ln in CHUNKS:
        pltpu.make_async_copy(
            table_hbm.at[idx_v.at[b, pl.ds(off, ln)]],
            rows_v.at[buf, pl.ds(off, ln)],
            sems[buf],
        ).wait()

    def reduce_into(b, buf):
      def red_body(r, carry):
        out = []
        for k in range(DV):
          a = carry[k]
          a = a + rows_v[buf, r, pl.ds(k * LANES, LANES)]
          a = a + rows_v[buf, r + 1, pl.ds(k * LANES, LANES)]
          out.append(a)
        return tuple(out)

      zeros = tuple(jnp.zeros((LANES,), jnp.float32) for _ in range(DV))
      acc = lax.fori_loop(0, SEQ // 2, lambda r, c: red_body(2 * r, c), zeros)
      for k in range(DV):
        acc_v[b, pl.ds(k * LANES, LANES)] = acc[k]

    issue(0, 0)

    def pair_body(p, _):
      b0 = 2 * p
      issue(b0 + 1, 1)
      wait(b0, 0)
      reduce_into(b0, 0)

      @pl.when(p < B_PER_W // 2 - 1)
      def _():
        issue(b0 + 2, 0)

      wait(b0 + 1, 1)
      reduce_into(b0 + 1, 1)
      return 0

    lax.fori_loop(0, B_PER_W // 2, pair_body, 0)
    pltpu.sync_copy(acc_v, out_hbm.at[pl.ds(base, B_PER_W)])

  return sc_kernel(x, table)


def _tc_epilogue(sums, x, t0, W, b):
  def tc_kernel(sums_ref, x_ref, t0_ref, w_ref, b_ref, out_ref):
    n0 = jnp.sum((x_ref[...] == 0).astype(jnp.float32), axis=1, keepdims=True)
    mean = (sums_ref[...] - n0 * t0_ref[...]) * (1.0 / SEQ)
    out_ref[...] = (
        jnp.dot(mean, w_ref[...].T, preferred_element_type=jnp.float32)
        + b_ref[...]
    )

  return pl.pallas_call(
      tc_kernel,
      out_shape=jax.ShapeDtypeStruct((BATCH, NUM_CLASSES), jnp.float32),
  )(sums, x, t0, W, b)


def kernel(x, table, W, b):
  sums = _sc_pooled_sums(x, table)
  t0 = lax.slice(table, (0, 0), (1, D))
  return _tc_epilogue(sums, x, t0, W, b.reshape(1, NUM_CLASSES))
